# baseline (device time: 792763 ns/iter reference)
import jax
import jax.numpy as jnp
from jax import lax
from jax.experimental import pallas as pl
from jax.experimental.pallas import tpu as pltpu

N_DEV = 8
S_CHUNK = 512
S_FULL = N_DEV * S_CHUNK
D_MODEL = 1024
N_HEADS = 8
D_HEAD = 128
Q_BLK = 512
SCALE = 0.08838834764831843
NSLOT = 3


def kernel(x, Wq, Wo, Wk, Wv):
    x2 = x.reshape(S_CHUNK, D_MODEL)

    def body(x_ref, wq_ref, wo_ref, wk_ref, wv_ref, out_ref,
             q_hbm, k_hbm, v_hbm, attn_hbm,
             ag_comm, rs_comm, stage, qh, kh, vh, oh,
             ag_send, ag_recv, rs_send, rs_recv, local_sem):
        my = lax.axis_index("i")
        left = lax.rem(my - 1 + N_DEV, N_DEV)
        right = lax.rem(my + 1, N_DEV)

        barrier = pltpu.get_barrier_semaphore()
        for nbr in (left, right):
            pl.semaphore_signal(barrier, inc=1, device_id=(nbr,),
                                device_id_type=pl.DeviceIdType.MESH)
        pl.semaphore_wait(barrier, 2)

        def to_hbm(dst_hbm, r0, value):
            stage[...] = value
            cp = pltpu.make_async_copy(
                stage, dst_hbm.at[pl.ds(r0, S_CHUNK), :], local_sem.at[0])
            cp.start()
            cp.wait()

        def project(chunk_idx, xc):
            r0 = chunk_idx * S_CHUNK
            xb = xc.astype(jnp.bfloat16)
            to_hbm(q_hbm, r0, jnp.dot(xb, wq_ref[...].astype(jnp.bfloat16),
                                      preferred_element_type=jnp.float32))
            to_hbm(k_hbm, r0, jnp.dot(xb, wk_ref[...].astype(jnp.bfloat16),
                                      preferred_element_type=jnp.float32))
            to_hbm(v_hbm, r0, jnp.dot(xb, wv_ref[...].astype(jnp.bfloat16),
                                      preferred_element_type=jnp.float32))

        ag_comm[0, :, :] = x_ref[...]
        for h in range(N_DEV - 1):
            s_slot, r_slot = h % NSLOT, (h + 1) % NSLOT
            rdma = pltpu.make_async_remote_copy(
                src_ref=ag_comm.at[s_slot],
                dst_ref=ag_comm.at[r_slot],
                send_sem=ag_send.at[s_slot],
                recv_sem=ag_recv.at[r_slot],
                device_id=(right,),
                device_id_type=pl.DeviceIdType.MESH,
            )
            rdma.start()
            if h == 0:
                project(my, x_ref[...])
            else:
                c = lax.rem(my - h + 2 * N_DEV, N_DEV)
                project(c, ag_comm[s_slot, :, :])
            rdma.wait()
        project(lax.rem(my + 1, N_DEV),
                ag_comm[(N_DEV - 1) % NSLOT, :, :])

        def head_body(hd, carry):
            col = hd * D_HEAD
            cps = [
                pltpu.make_async_copy(
                    src.at[:, pl.ds(col, D_HEAD)], dst, local_sem.at[i])
                for i, (src, dst) in enumerate(
                    ((q_hbm, qh), (k_hbm, kh), (v_hbm, vh)))
            ]
            for cp in cps:
                cp.start()
            for cp in cps:
                cp.wait()

            def q_body(qb, carry2):
                r0 = qb * Q_BLK
                qblk = qh[pl.ds(r0, Q_BLK), :].astype(jnp.bfloat16)
                s = lax.dot_general(
                    qblk, kh[...].astype(jnp.bfloat16),
                    (((1,), (1,)), ((), ())),
                    preferred_element_type=jnp.float32) * SCALE
                m = jnp.max(s, axis=1, keepdims=True)
                p = jnp.exp(s - m)
                denom = jnp.sum(p, axis=1, keepdims=True)
                oh[pl.ds(r0, Q_BLK), :] = jnp.dot(
                    p.astype(jnp.bfloat16), vh[...].astype(jnp.bfloat16),
                    preferred_element_type=jnp.float32) / denom
                return carry2

            lax.fori_loop(0, S_FULL // Q_BLK, q_body, 0)
            cp = pltpu.make_async_copy(
                oh, attn_hbm.at[:, pl.ds(col, D_HEAD)], local_sem.at[3])
            cp.start()
            cp.wait()
            return carry

        lax.fori_loop(0, N_HEADS, head_body, 0)

        def chunk_partial(c):
            cp = pltpu.make_async_copy(
                attn_hbm.at[pl.ds(c * S_CHUNK, S_CHUNK), :], stage,
                local_sem.at[0])
            cp.start()
            cp.wait()
            return jnp.dot(stage[...].astype(jnp.bfloat16),
                           wo_ref[...].astype(jnp.bfloat16),
                           preferred_element_type=jnp.float32)

        prev = None
        for s in range(N_DEV - 1):
            s_slot, r_slot = s % NSLOT, (s + 1) % NSLOT
            c = lax.rem(my - 1 - s + 2 * N_DEV, N_DEV)
            pc = chunk_partial(c)
            if prev is not None:
                prev.wait()
            if s == 0:
                rs_comm[s_slot, :, :] = pc
            else:
                rs_comm[s_slot, :, :] = pc + rs_comm[s_slot, :, :]
            rdma = pltpu.make_async_remote_copy(
                src_ref=rs_comm.at[s_slot],
                dst_ref=rs_comm.at[r_slot],
                send_sem=rs_send.at[s_slot],
                recv_sem=rs_recv.at[r_slot],
                device_id=(right,),
                device_id_type=pl.DeviceIdType.MESH,
            )
            rdma.start()
            prev = rdma

        pc = chunk_partial(my)
        prev.wait()
        out_ref[...] = pc + rs_comm[(N_DEV - 1) % NSLOT, :, :]

    hbm_scratch = jax.ShapeDtypeStruct((S_FULL, D_MODEL), jnp.float32)
    out = pl.pallas_call(
        body,
        out_shape=(
            jax.ShapeDtypeStruct((S_CHUNK, D_MODEL), jnp.float32),
            hbm_scratch,
            hbm_scratch,
            hbm_scratch,
            hbm_scratch,
        ),
        in_specs=[pl.BlockSpec(memory_space=pltpu.VMEM)] * 5,
        out_specs=(
            pl.BlockSpec(memory_space=pltpu.VMEM),
            pl.BlockSpec(memory_space=pltpu.HBM),
            pl.BlockSpec(memory_space=pltpu.HBM),
            pl.BlockSpec(memory_space=pltpu.HBM),
            pl.BlockSpec(memory_space=pltpu.HBM),
        ),
        scratch_shapes=[
            pltpu.VMEM((NSLOT, S_CHUNK, D_MODEL), jnp.float32),
            pltpu.VMEM((NSLOT, S_CHUNK, D_MODEL), jnp.float32),
            pltpu.VMEM((S_CHUNK, D_MODEL), jnp.float32),
            pltpu.VMEM((S_FULL, D_HEAD), jnp.float32),
            pltpu.VMEM((S_FULL, D_HEAD), jnp.float32),
            pltpu.VMEM((S_FULL, D_HEAD), jnp.float32),
            pltpu.VMEM((S_FULL, D_HEAD), jnp.float32),
            pltpu.SemaphoreType.DMA((NSLOT,)),
            pltpu.SemaphoreType.DMA((NSLOT,)),
            pltpu.SemaphoreType.DMA((NSLOT,)),
            pltpu.SemaphoreType.DMA((NSLOT,)),
            pltpu.SemaphoreType.DMA((4,)),
        ],
        compiler_params=pltpu.CompilerParams(
            collective_id=0,
            vmem_limit_bytes=56 * 1024 * 1024,
        ),
    )(x2, Wq, Wo, Wk, Wv)
    return out[0].reshape(1, S_CHUNK, D_MODEL)


# device time: 398907 ns/iter; 1.9873x vs baseline; 1.9873x over previous
import jax
import jax.numpy as jnp
from jax import lax
from jax.experimental import pallas as pl
from jax.experimental.pallas import tpu as pltpu

N_DEV = 8
S_CHUNK = 512
S_FULL = N_DEV * S_CHUNK
D_MODEL = 1024
N_HEADS = 8
D_HEAD = 128
Q_BLK = 512
SCALE = 0.08838834764831843
NSLOT = 3
BF16 = jnp.bfloat16


def kernel(x, Wq, Wo, Wk, Wv):
    x2 = x.reshape(S_CHUNK, D_MODEL)

    def body(x_ref, wq_ref, wo_ref, wk_ref, wv_ref, out_ref,
             q_hbm, k_hbm, v_hbm, attn_hbm,
             ag_comm, rs_comm, stage, qh, kh, vh, oh,
             ag_send, ag_recv, rs_send, rs_recv, local_sem):
        my = lax.axis_index("i")
        left = lax.rem(my - 1 + N_DEV, N_DEV)
        right = lax.rem(my + 1, N_DEV)

        barrier = pltpu.get_barrier_semaphore()
        for nbr in (left, right):
            pl.semaphore_signal(barrier, inc=1, device_id=(nbr,),
                                device_id_type=pl.DeviceIdType.MESH)
        pl.semaphore_wait(barrier, 2)

        def to_hbm(dst_hbm, r0, value):
            stage[...] = value
            cp = pltpu.make_async_copy(
                stage, dst_hbm.at[pl.ds(r0, S_CHUNK), :], local_sem.at[0])
            cp.start()
            cp.wait()

        def project(chunk_idx, xb):
            r0 = chunk_idx * S_CHUNK
            to_hbm(q_hbm, r0, jnp.dot(xb, wq_ref[...].astype(BF16),
                                      preferred_element_type=jnp.float32
                                      ).astype(BF16))
            to_hbm(k_hbm, r0, jnp.dot(xb, wk_ref[...].astype(BF16),
                                      preferred_element_type=jnp.float32
                                      ).astype(BF16))
            to_hbm(v_hbm, r0, jnp.dot(xb, wv_ref[...].astype(BF16),
                                      preferred_element_type=jnp.float32
                                      ).astype(BF16))

        ag_comm[0, :, :] = x_ref[...].astype(BF16)
        for h in range(N_DEV - 1):
            s_slot, r_slot = h % NSLOT, (h + 1) % NSLOT
            rdma = pltpu.make_async_remote_copy(
                src_ref=ag_comm.at[s_slot],
                dst_ref=ag_comm.at[r_slot],
                send_sem=ag_send.at[s_slot],
                recv_sem=ag_recv.at[r_slot],
                device_id=(right,),
                device_id_type=pl.DeviceIdType.MESH,
            )
            rdma.start()
            if h == 0:
                project(my, x_ref[...].astype(BF16))
            else:
                c = lax.rem(my - h + 2 * N_DEV, N_DEV)
                project(c, ag_comm[s_slot, :, :])
            rdma.wait()
        project(lax.rem(my + 1, N_DEV),
                ag_comm[(N_DEV - 1) % NSLOT, :, :])

        def head_body(hd, carry):
            col = hd * D_HEAD
            cps = [
                pltpu.make_async_copy(
                    src.at[:, pl.ds(col, D_HEAD)], dst, local_sem.at[i])
                for i, (src, dst) in enumerate(
                    ((q_hbm, qh), (k_hbm, kh), (v_hbm, vh)))
            ]
            for cp in cps:
                cp.start()
            for cp in cps:
                cp.wait()
            kb = kh[...]
            vb = vh[...]
            for qb in range(S_FULL // Q_BLK):
                r0 = qb * Q_BLK
                s = lax.dot_general(
                    qh[pl.ds(r0, Q_BLK), :], kb, (((1,), (1,)), ((), ())),
                    preferred_element_type=jnp.float32) * SCALE
                p = jnp.exp(s)
                denom = jnp.sum(p, axis=1, keepdims=True)
                o = jnp.dot(p.astype(BF16), vb,
                            preferred_element_type=jnp.float32) / denom
                oh[pl.ds(r0, Q_BLK), :] = o.astype(BF16)
            cp = pltpu.make_async_copy(
                oh, attn_hbm.at[:, pl.ds(col, D_HEAD)], local_sem.at[3])
            cp.start()
            cp.wait()
            return carry

        lax.fori_loop(0, N_HEADS, head_body, 0)

        def chunk_partial(c):
            cp = pltpu.make_async_copy(
                attn_hbm.at[pl.ds(c * S_CHUNK, S_CHUNK), :], stage,
                local_sem.at[0])
            cp.start()
            cp.wait()
            return jnp.dot(stage[...], wo_ref[...].astype(BF16),
                           preferred_element_type=jnp.float32)

        prev = None
        for s in range(N_DEV - 1):
            s_slot, r_slot = s % NSLOT, (s + 1) % NSLOT
            c = lax.rem(my - 1 - s + 2 * N_DEV, N_DEV)
            pc = chunk_partial(c)
            if prev is not None:
                prev.wait()
            if s == 0:
                rs_comm[s_slot, :, :] = pc.astype(BF16)
            else:
                rs_comm[s_slot, :, :] = (
                    pc + rs_comm[s_slot, :, :].astype(jnp.float32)
                ).astype(BF16)
            rdma = pltpu.make_async_remote_copy(
                src_ref=rs_comm.at[s_slot],
                dst_ref=rs_comm.at[r_slot],
                send_sem=rs_send.at[s_slot],
                recv_sem=rs_recv.at[r_slot],
                device_id=(right,),
                device_id_type=pl.DeviceIdType.MESH,
            )
            rdma.start()
            prev = rdma

        pc = chunk_partial(my)
        prev.wait()
        out_ref[...] = pc + rs_comm[(N_DEV - 1) % NSLOT, :, :].astype(
            jnp.float32)

    hbm_scratch = jax.ShapeDtypeStruct((S_FULL, D_MODEL), BF16)
    out = pl.pallas_call(
        body,
        out_shape=(
            jax.ShapeDtypeStruct((S_CHUNK, D_MODEL), jnp.float32),
            hbm_scratch,
            hbm_scratch,
            hbm_scratch,
            hbm_scratch,
        ),
        in_specs=[pl.BlockSpec(memory_space=pltpu.VMEM)] * 5,
        out_specs=(
            pl.BlockSpec(memory_space=pltpu.VMEM),
            pl.BlockSpec(memory_space=pltpu.HBM),
            pl.BlockSpec(memory_space=pltpu.HBM),
            pl.BlockSpec(memory_space=pltpu.HBM),
            pl.BlockSpec(memory_space=pltpu.HBM),
        ),
        scratch_shapes=[
            pltpu.VMEM((NSLOT, S_CHUNK, D_MODEL), BF16),
            pltpu.VMEM((NSLOT, S_CHUNK, D_MODEL), BF16),
            pltpu.VMEM((S_CHUNK, D_MODEL), BF16),
            pltpu.VMEM((S_FULL, D_HEAD), BF16),
            pltpu.VMEM((S_FULL, D_HEAD), BF16),
            pltpu.VMEM((S_FULL, D_HEAD), BF16),
            pltpu.VMEM((S_FULL, D_HEAD), BF16),
            pltpu.SemaphoreType.DMA((NSLOT,)),
            pltpu.SemaphoreType.DMA((NSLOT,)),
            pltpu.SemaphoreType.DMA((NSLOT,)),
            pltpu.SemaphoreType.DMA((NSLOT,)),
            pltpu.SemaphoreType.DMA((4,)),
        ],
        compiler_params=pltpu.CompilerParams(
            collective_id=0,
            vmem_limit_bytes=56 * 1024 * 1024,
        ),
    )(x2, Wq, Wo, Wk, Wv)
    return out[0].reshape(1, S_CHUNK, D_MODEL)


# device time: 316264 ns/iter; 2.5066x vs baseline; 1.2613x over previous
import jax
import jax.numpy as jnp
from jax import lax
from jax.experimental import pallas as pl
from jax.experimental.pallas import tpu as pltpu

N_DEV = 8
S_CHUNK = 512
S_FULL = N_DEV * S_CHUNK
D_MODEL = 1024
N_HEADS = 8
D_HEAD = 128
Q_BLK = 256
SCALE = 0.08838834764831843
NSLOT = 3
BF16 = jnp.bfloat16


def kernel(x, Wq, Wo, Wk, Wv):
    x2 = x.reshape(S_CHUNK, D_MODEL)

    def body(x_ref, wq_ref, wo_ref, wk_ref, wv_ref, out_ref,
             q_full, k_full, v_full, ag_comm, rs_comm, attn_buf,
             ag_send, ag_recv, rs_send, rs_recv):
        my = lax.axis_index("i")
        left = lax.rem(my - 1 + N_DEV, N_DEV)
        right = lax.rem(my + 1, N_DEV)

        barrier = pltpu.get_barrier_semaphore()
        for nbr in (left, right):
            pl.semaphore_signal(barrier, inc=1, device_id=(nbr,),
                                device_id_type=pl.DeviceIdType.MESH)
        pl.semaphore_wait(barrier, 2)

        def project(chunk_idx, xb):
            r0 = chunk_idx * S_CHUNK
            for w_ref, dst in ((wq_ref, q_full), (wk_ref, k_full),
                               (wv_ref, v_full)):
                dst[pl.ds(r0, S_CHUNK), :] = jnp.dot(
                    xb, w_ref[...].astype(BF16),
                    preferred_element_type=jnp.float32).astype(BF16)

        ag_comm[0, :, :] = x_ref[...].astype(BF16)
        for h in range(N_DEV - 1):
            s_slot, r_slot = h % NSLOT, (h + 1) % NSLOT
            rdma = pltpu.make_async_remote_copy(
                src_ref=ag_comm.at[s_slot],
                dst_ref=ag_comm.at[r_slot],
                send_sem=ag_send.at[s_slot],
                recv_sem=ag_recv.at[r_slot],
                device_id=(right,),
                device_id_type=pl.DeviceIdType.MESH,
            )
            rdma.start()
            if h == 0:
                project(my, x_ref[...].astype(BF16))
            else:
                c = lax.rem(my - h + 2 * N_DEV, N_DEV)
                project(c, ag_comm[s_slot, :, :])
            rdma.wait()
        project(lax.rem(my + 1, N_DEV),
                ag_comm[(N_DEV - 1) % NSLOT, :, :])

        def chunk_partial(c):
            r0c = c * S_CHUNK

            def head_body(hd, carry):
                col = hd * D_HEAD
                kb = k_full[:, pl.ds(col, D_HEAD)]
                vb = v_full[:, pl.ds(col, D_HEAD)]
                for sub in range(S_CHUNK // Q_BLK):
                    r0 = r0c * 1 + sub * Q_BLK
                    s = lax.dot_general(
                        q_full[pl.ds(r0, Q_BLK), pl.ds(col, D_HEAD)], kb,
                        (((1,), (1,)), ((), ())),
                        preferred_element_type=jnp.float32) * SCALE
                    p = jnp.exp(s)
                    denom = jnp.sum(p, axis=1, keepdims=True)
                    o = jnp.dot(p.astype(BF16), vb,
                                preferred_element_type=jnp.float32) / denom
                    attn_buf[pl.ds(sub * Q_BLK, Q_BLK),
                             pl.ds(col, D_HEAD)] = o.astype(BF16)
                return carry

            lax.fori_loop(0, N_HEADS, head_body, 0)
            return jnp.dot(attn_buf[...], wo_ref[...].astype(BF16),
                           preferred_element_type=jnp.float32)

        prev = None
        for s in range(N_DEV - 1):
            s_slot, r_slot = s % NSLOT, (s + 1) % NSLOT
            c = lax.rem(my - 1 - s + 2 * N_DEV, N_DEV)
            pc = chunk_partial(c)
            if prev is not None:
                prev.wait()
            if s == 0:
                rs_comm[s_slot, :, :] = pc.astype(BF16)
            else:
                rs_comm[s_slot, :, :] = (
                    pc + rs_comm[s_slot, :, :].astype(jnp.float32)
                ).astype(BF16)
            rdma = pltpu.make_async_remote_copy(
                src_ref=rs_comm.at[s_slot],
                dst_ref=rs_comm.at[r_slot],
                send_sem=rs_send.at[s_slot],
                recv_sem=rs_recv.at[r_slot],
                device_id=(right,),
                device_id_type=pl.DeviceIdType.MESH,
            )
            rdma.start()
            prev = rdma

        pc = chunk_partial(my)
        prev.wait()
        out_ref[...] = pc + rs_comm[(N_DEV - 1) % NSLOT, :, :].astype(
            jnp.float32)

    out = pl.pallas_call(
        body,
        out_shape=jax.ShapeDtypeStruct((S_CHUNK, D_MODEL), jnp.float32),
        in_specs=[pl.BlockSpec(memory_space=pltpu.VMEM)] * 5,
        out_specs=pl.BlockSpec(memory_space=pltpu.VMEM),
        scratch_shapes=[
            pltpu.VMEM((S_FULL, D_MODEL), BF16),
            pltpu.VMEM((S_FULL, D_MODEL), BF16),
            pltpu.VMEM((S_FULL, D_MODEL), BF16),
            pltpu.VMEM((NSLOT, S_CHUNK, D_MODEL), BF16),
            pltpu.VMEM((NSLOT, S_CHUNK, D_MODEL), BF16),
            pltpu.VMEM((S_CHUNK, D_MODEL), BF16),
            pltpu.SemaphoreType.DMA((NSLOT,)),
            pltpu.SemaphoreType.DMA((NSLOT,)),
            pltpu.SemaphoreType.DMA((NSLOT,)),
            pltpu.SemaphoreType.DMA((NSLOT,)),
        ],
        compiler_params=pltpu.CompilerParams(
            collective_id=0,
            vmem_limit_bytes=61 * 1024 * 1024,
        ),
    )(x2, Wq, Wo, Wk, Wv)
    return out.reshape(1, S_CHUNK, D_MODEL)


# device time: 285274 ns/iter; 2.7790x vs baseline; 1.1086x over previous
import jax
import jax.numpy as jnp
from jax import lax
from jax.experimental import pallas as pl
from jax.experimental.pallas import tpu as pltpu

N_DEV = 8
S_CHUNK = 512
S_FULL = N_DEV * S_CHUNK
D_MODEL = 1024
N_HEADS = 8
D_HEAD = 128
Q_BLK = 512
SCALE = 0.08838834764831843
NSLOT = 3
BF16 = jnp.bfloat16


def kernel(x, Wq, Wo, Wk, Wv):
    x2 = x.reshape(S_CHUNK, D_MODEL)

    def body(x_ref, wq_ref, wo_ref, wk_ref, wv_ref, out_ref,
             q_full, k_full, v_full, ag_comm, rs_comm, attn_buf,
             ag_send, ag_recv, rs_send, rs_recv):
        my = lax.axis_index("i")
        left = lax.rem(my - 1 + N_DEV, N_DEV)
        right = lax.rem(my + 1, N_DEV)

        barrier = pltpu.get_barrier_semaphore()
        for nbr in (left, right):
            pl.semaphore_signal(barrier, inc=1, device_id=(nbr,),
                                device_id_type=pl.DeviceIdType.MESH)
        pl.semaphore_wait(barrier, 2)

        def project(chunk_idx, xb):
            r0 = chunk_idx * S_CHUNK
            for w_ref, dst in ((wq_ref, q_full), (wk_ref, k_full),
                               (wv_ref, v_full)):
                dst[pl.ds(r0, S_CHUNK), :] = jnp.dot(
                    xb, w_ref[...].astype(BF16),
                    preferred_element_type=jnp.float32).astype(BF16)

        ag_comm[0, :, :] = x_ref[...].astype(BF16)
        for h in range(N_DEV - 1):
            s_slot, r_slot = h % NSLOT, (h + 1) % NSLOT
            rdma = pltpu.make_async_remote_copy(
                src_ref=ag_comm.at[s_slot],
                dst_ref=ag_comm.at[r_slot],
                send_sem=ag_send.at[s_slot],
                recv_sem=ag_recv.at[r_slot],
                device_id=(right,),
                device_id_type=pl.DeviceIdType.MESH,
            )
            rdma.start()
            if h == 0:
                project(my, x_ref[...].astype(BF16))
            else:
                c = lax.rem(my - h + 2 * N_DEV, N_DEV)
                project(c, ag_comm[s_slot, :, :])
            rdma.wait()
        project(lax.rem(my + 1, N_DEV),
                ag_comm[(N_DEV - 1) % NSLOT, :, :])

        def chunk_partial(c):
            r0c = c * S_CHUNK

            def head_body(hd, carry):
                col = hd * D_HEAD
                kb = k_full[:, pl.ds(col, D_HEAD)]
                vb = v_full[:, pl.ds(col, D_HEAD)]
                for sub in range(S_CHUNK // Q_BLK):
                    r0 = r0c + sub * Q_BLK
                    s = lax.dot_general(
                        q_full[pl.ds(r0, Q_BLK), pl.ds(col, D_HEAD)], kb,
                        (((1,), (1,)), ((), ())),
                        preferred_element_type=jnp.float32) * SCALE
                    p = jnp.exp(s)
                    denom = jnp.sum(p, axis=1, keepdims=True)
                    o = jnp.dot(p.astype(BF16), vb,
                                preferred_element_type=jnp.float32) / denom
                    attn_buf[pl.ds(sub * Q_BLK, Q_BLK),
                             pl.ds(col, D_HEAD)] = o.astype(BF16)
                return carry

            lax.fori_loop(0, N_HEADS, head_body, 0)
            return jnp.dot(attn_buf[...], wo_ref[...].astype(BF16),
                           preferred_element_type=jnp.float32)

        prev = None
        for s in range(N_DEV - 1):
            s_slot, r_slot = s % NSLOT, (s + 1) % NSLOT
            c = lax.rem(my - 1 - s + 2 * N_DEV, N_DEV)
            pc = chunk_partial(c)
            if prev is not None:
                prev.wait()
            if s == 0:
                rs_comm[s_slot, :, :] = pc.astype(BF16)
            else:
                rs_comm[s_slot, :, :] = (
                    pc + rs_comm[s_slot, :, :].astype(jnp.float32)
                ).astype(BF16)
            rdma = pltpu.make_async_remote_copy(
                src_ref=rs_comm.at[s_slot],
                dst_ref=rs_comm.at[r_slot],
                send_sem=rs_send.at[s_slot],
                recv_sem=rs_recv.at[r_slot],
                device_id=(right,),
                device_id_type=pl.DeviceIdType.MESH,
            )
            rdma.start()
            prev = rdma

        pc = chunk_partial(my)
        prev.wait()
        out_ref[...] = pc + rs_comm[(N_DEV - 1) % NSLOT, :, :].astype(
            jnp.float32)

    out = pl.pallas_call(
        body,
        out_shape=jax.ShapeDtypeStruct((S_CHUNK, D_MODEL), jnp.float32),
        in_specs=[pl.BlockSpec(memory_space=pltpu.VMEM)] * 5,
        out_specs=pl.BlockSpec(memory_space=pltpu.VMEM),
        scratch_shapes=[
            pltpu.VMEM((S_FULL, D_MODEL), BF16),
            pltpu.VMEM((S_FULL, D_MODEL), BF16),
            pltpu.VMEM((S_FULL, D_MODEL), BF16),
            pltpu.VMEM((NSLOT, S_CHUNK, D_MODEL), BF16),
            pltpu.VMEM((NSLOT, S_CHUNK, D_MODEL), BF16),
            pltpu.VMEM((S_CHUNK, D_MODEL), BF16),
            pltpu.SemaphoreType.DMA((NSLOT,)),
            pltpu.SemaphoreType.DMA((NSLOT,)),
            pltpu.SemaphoreType.DMA((NSLOT,)),
            pltpu.SemaphoreType.DMA((NSLOT,)),
        ],
        compiler_params=pltpu.CompilerParams(
            collective_id=0,
            vmem_limit_bytes=61 * 1024 * 1024,
        ),
    )(x2, Wq, Wo, Wk, Wv)
    return out.reshape(1, S_CHUNK, D_MODEL)
